# baseline (speedup 1.0000x reference)
import functools

import jax
import jax.numpy as jnp
from jax import lax
from jax.experimental import pallas as pl
from jax.experimental.pallas import tpu as pltpu
from jax.experimental.pallas import tpu_sc as plsc

BATCH = 4096
MAX_LEN = 200
EMB = 32
NC = 2
NS = 16
NW = NC * NS
BPW = BATCH // NW
UNROLL = 4


def _body(
    idxT_hbm, table_hbm, pos_hbm, out_hbm, idx_v, bufs, pos_v, sem_a, sem_b, sem_oa, sem_ob
):
    wid = lax.axis_index("s") * NC + lax.axis_index("c")
    pltpu.sync_copy(idxT_hbm.at[:, pl.ds(wid * BPW, BPW)], idx_v)
    pltpu.sync_copy(pos_hbm, pos_v)

    def gather(l, slot, sem):
        pltpu.async_copy(table_hbm.at[idx_v.at[l]], bufs.at[slot], sem)

    def consume(l, slot, sem, osem):
        pltpu.make_async_copy(
            table_hbm.at[idx_v.at[l]], bufs.at[slot], sem
        ).wait()
        pos_h = [pos_v[l, pl.ds(16 * h, 16)] for h in range(2)]

        def tok_body(b4, carry2):
            for u in range(UNROLL):
                b = b4 * UNROLL + u
                for h in range(2):
                    bufs[slot, b, pl.ds(16 * h, 16)] = (
                        bufs[slot, b, pl.ds(16 * h, 16)] + pos_h[h]
                    )
            return carry2

        lax.fori_loop(0, BPW // UNROLL, tok_body, 0)
        pltpu.async_copy(bufs.at[slot], out_hbm.at[l].at[wid], osem)

    def drain_out(l, slot, osem):
        pltpu.make_async_copy(
            bufs.at[slot], out_hbm.at[l].at[wid], osem
        ).wait()

    gather(0, 0, sem_a)
    gather(1, 1, sem_b)

    def pos_body(i, carry):
        l0 = 2 * i
        consume(l0, 0, sem_a, sem_oa)
        consume(l0 + 1, 1, sem_b, sem_ob)
        drain_out(l0, 0, sem_oa)
        gather(jnp.minimum(l0 + 2, MAX_LEN - 1), 0, sem_a)
        drain_out(l0 + 1, 1, sem_ob)
        gather(jnp.minimum(l0 + 3, MAX_LEN - 1), 1, sem_b)
        return carry

    lax.fori_loop(0, MAX_LEN // 2, pos_body, 0)
    pltpu.make_async_copy(
        table_hbm.at[idx_v.at[MAX_LEN - 1]], bufs.at[0], sem_a
    ).wait()
    pltpu.make_async_copy(
        table_hbm.at[idx_v.at[MAX_LEN - 1]], bufs.at[1], sem_b
    ).wait()


@jax.jit
def _run(idxT, token_table, pos_table):
    mesh = plsc.VectorSubcoreMesh(core_axis_name="c", subcore_axis_name="s")
    k = functools.partial(
        pl.kernel,
        mesh=mesh,
        out_type=jax.ShapeDtypeStruct((MAX_LEN, NW, BPW, EMB), jnp.float32),
        scratch_types=[
            pltpu.VMEM((MAX_LEN, BPW), jnp.int32),
            pltpu.VMEM((2, BPW, EMB), jnp.float32),
            pltpu.VMEM((MAX_LEN, EMB), jnp.float32),
            pltpu.SemaphoreType.DMA,
            pltpu.SemaphoreType.DMA,
            pltpu.SemaphoreType.DMA,
            pltpu.SemaphoreType.DMA,
        ],
        compiler_params=pltpu.CompilerParams(
            use_tc_tiling_on_sc=False, needs_layout_passes=False
        ),
    )(_body)
    return k(idxT, token_table, pos_table)


def kernel(batch_seqs, token_table, pos_table):
    out4 = _run(batch_seqs.T, token_table, pos_table)
    return out4.transpose(1, 2, 0, 3).reshape(BATCH, MAX_LEN, EMB)


# speedup vs baseline: 1.0042x; 1.0042x over previous
import functools

import jax
import jax.numpy as jnp
from jax import lax
from jax.experimental import pallas as pl
from jax.experimental.pallas import tpu as pltpu
from jax.experimental.pallas import tpu_sc as plsc

BATCH = 4096
MAX_LEN = 200
EMB = 32
NC = 2
NS = 16
NW = NC * NS
BPW = BATCH // NW
UNROLL = 4


def _body(
    idxT_hbm, table_hbm, pos_hbm, out_hbm, idx_v, bufs, pos_v, sem_a, sem_b
):
    wid = lax.axis_index("s") * NC + lax.axis_index("c")
    pltpu.sync_copy(idxT_hbm.at[:, pl.ds(wid * BPW, BPW)], idx_v)
    pltpu.sync_copy(pos_hbm, pos_v)

    def gather(l, slot, sem):
        pltpu.async_copy(table_hbm.at[idx_v.at[l]], bufs.at[slot], sem)

    def consume(l, slot, sem):
        pltpu.make_async_copy(
            table_hbm.at[idx_v.at[l]], bufs.at[slot], sem
        ).wait()
        pos_h = [pos_v[l, pl.ds(16 * h, 16)] for h in range(2)]

        def tok_body(b4, carry2):
            for u in range(UNROLL):
                b = b4 * UNROLL + u
                for h in range(2):
                    bufs[slot, b, pl.ds(16 * h, 16)] = (
                        bufs[slot, b, pl.ds(16 * h, 16)] + pos_h[h]
                    )
            return carry2

        lax.fori_loop(0, BPW // UNROLL, tok_body, 0)
        pltpu.sync_copy(bufs.at[slot], out_hbm.at[l].at[wid])

    gather(0, 0, sem_a)

    def pos_body(i, carry):
        l0 = 2 * i
        gather(l0 + 1, 1, sem_b)
        consume(l0, 0, sem_a)
        gather(jnp.minimum(l0 + 2, MAX_LEN - 1), 0, sem_a)
        consume(l0 + 1, 1, sem_b)
        return carry

    lax.fori_loop(0, MAX_LEN // 2, pos_body, 0)
    pltpu.make_async_copy(
        table_hbm.at[idx_v.at[MAX_LEN - 1]], bufs.at[0], sem_a
    ).wait()


@jax.jit
def _run(idxT, token_table, pos_table):
    mesh = plsc.VectorSubcoreMesh(core_axis_name="c", subcore_axis_name="s")
    k = functools.partial(
        pl.kernel,
        mesh=mesh,
        out_type=jax.ShapeDtypeStruct((MAX_LEN, NW, BPW, EMB), jnp.float32),
        scratch_types=[
            pltpu.VMEM((MAX_LEN, BPW), jnp.int32),
            pltpu.VMEM((2, BPW, EMB), jnp.float32),
            pltpu.VMEM((MAX_LEN, EMB), jnp.float32),
            pltpu.SemaphoreType.DMA,
            pltpu.SemaphoreType.DMA,
        ],
        compiler_params=pltpu.CompilerParams(
            use_tc_tiling_on_sc=False, needs_layout_passes=False
        ),
    )(_body)
    return k(idxT, token_table, pos_table)


def kernel(batch_seqs, token_table, pos_table):
    out4 = _run(batch_seqs.T, token_table, pos_table)
    return out4.transpose(1, 2, 0, 3).reshape(BATCH, MAX_LEN, EMB)
